# baseline (speedup 1.0000x reference)
import functools

import jax
import jax.numpy as jnp
from jax import lax
from jax.experimental import pallas as pl
from jax.experimental.pallas import tpu as pltpu
from jax.experimental.pallas import tpu_sc as plsc

S, H, E, K = 2048, 1024, 8, 2
F = 4 * H
BM = 128
NA = S * K
T = NA // BM + E
P = T * BM

NC, NS = 2, 16
NW = NC * NS

_PREC_ROUTER = lax.Precision.DEFAULT


def _router_body(x_ref, wg_ref, w_ref, i_ref):
    logits = jnp.dot(x_ref[...], wg_ref[...],
                     preferred_element_type=jnp.float32,
                     precision=_PREC_ROUTER)
    lane = lax.broadcasted_iota(jnp.int32, logits.shape, 1)
    valid = lane < E
    logits = jnp.where(valid, logits, -1e30)
    m = jnp.max(logits, axis=-1, keepdims=True)
    ex = jnp.where(valid, jnp.exp(logits - m), 0.0)
    probs = ex / jnp.sum(ex, axis=-1, keepdims=True)
    m1 = jnp.max(probs, axis=-1, keepdims=True)
    i1 = jnp.min(jnp.where(probs == m1, lane, E), axis=-1, keepdims=True)
    probs2 = jnp.where(lane == i1, -1.0, probs)
    m2 = jnp.max(probs2, axis=-1, keepdims=True)
    i2 = jnp.min(jnp.where(probs2 == m2, lane, E), axis=-1, keepdims=True)
    w_ref[...] = jnp.where(lane == 0, m1, 0.0) + jnp.where(lane == 1, m2, 0.0)
    i_ref[...] = jnp.where(lane == 0, i1, 0) + jnp.where(lane == 1, i2, 0)


_ROUTER_BS = 512

_router = pl.pallas_call(
    _router_body,
    grid=(S // _ROUTER_BS,),
    in_specs=[
        pl.BlockSpec((_ROUTER_BS, H), lambda i: (i, 0)),
        pl.BlockSpec((H, 128), lambda i: (0, 0)),
    ],
    out_specs=[
        pl.BlockSpec((_ROUTER_BS, 128), lambda i: (i, 0)),
        pl.BlockSpec((_ROUTER_BS, 128), lambda i: (i, 0)),
    ],
    out_shape=[
        jax.ShapeDtypeStruct((S, 128), jnp.float32),
        jax.ShapeDtypeStruct((S, 128), jnp.int32),
    ],
)


_MB = 128
_NMB = S // _MB


def _meta_body(i_ref, pos_ref, aux_ref):
    ltri = (lax.broadcasted_iota(jnp.int32, (_MB, _MB), 1)
            < lax.broadcasted_iota(jnp.int32, (_MB, _MB), 0)
            ).astype(jnp.bfloat16)

    def blk(b):
        sl = pl.ds(b * _MB, _MB)
        lane = lax.broadcasted_iota(jnp.int32, (_MB, 128), 1)
        e0 = i_ref[sl, 0:1]
        e1 = i_ref[sl, 1:2]
        oh0 = (lane == e0).astype(jnp.float32)
        oh1 = (lane == e1).astype(jnp.float32)
        return sl, lane, oh0, oh1, oh0 + oh1

    cnt = jnp.zeros((1, 128), jnp.float32)
    for b in range(_NMB):
        _, _, _, _, mb = blk(b)
        cnt = cnt + jnp.sum(mb, axis=0, keepdims=True)

    blocks = jnp.floor((cnt + (BM - 1)) * (1.0 / BM))
    ut = (lax.broadcasted_iota(jnp.int32, (128, 128), 0)
          <= lax.broadcasted_iota(jnp.int32, (128, 128), 1)
          ).astype(jnp.bfloat16)
    bcum = jnp.dot(blocks.astype(jnp.bfloat16), ut,
                   preferred_element_type=jnp.float32)
    bstart = (bcum - blocks) * BM
    aux_ref[...] = jnp.broadcast_to(bcum, (8, 128)).astype(jnp.int32)

    tot = jnp.zeros((1, 128), jnp.float32)
    for b in range(_NMB):
        sl, lane, oh0, oh1, mb = blk(b)
        c = jnp.dot(ltri, mb.astype(jnp.bfloat16),
                    preferred_element_type=jnp.float32) + tot
        rank0 = jnp.sum(oh0 * c, axis=1, keepdims=True)
        rank1 = jnp.sum(oh1 * (c + oh0), axis=1, keepdims=True)
        s0 = jnp.sum(oh0 * bstart, axis=1, keepdims=True)
        s1 = jnp.sum(oh1 * bstart, axis=1, keepdims=True)
        pos0 = (s0 + rank0).astype(jnp.int32)
        pos1 = (s1 + rank1).astype(jnp.int32)
        pos_ref[sl, :] = (jnp.where(lane == 0, pos0, 0)
                          + jnp.where(lane == 1, pos1, 0))
        tot = tot + jnp.sum(mb, axis=0, keepdims=True)


_meta = pl.pallas_call(
    _meta_body,
    in_specs=[pl.BlockSpec((S, 128), lambda: (0, 0))],
    out_specs=[
        pl.BlockSpec((S, 128), lambda: (0, 0)),
        pl.BlockSpec((8, 128), lambda: (0, 0)),
    ],
    out_shape=[
        jax.ShapeDtypeStruct((S, 128), jnp.int32),
        jax.ShapeDtypeStruct((8, 128), jnp.int32),
    ],
)


_TCH = 16
_TNCH = (S // NW) // _TCH


def _iota16():
    return lax.iota(jnp.int32, 16)


def _sc_dispatch_body(src_hbm, i0_hbm, i1_hbm, out_hbm, i0_v, i1_v,
                      x0, x1, s0, s1):
    wid = lax.axis_index("s") * NC + lax.axis_index("c")
    tok_per_w = S // NW
    base = wid * tok_per_w
    xbufs = [(x0, s0), (x1, s1)]
    pltpu.sync_copy(i0_hbm.at[pl.ds(base, tok_per_w)], i0_v)
    pltpu.sync_copy(i1_hbm.at[pl.ds(base, tok_per_w)], i1_v)

    scats = [None] * _TNCH
    for ci in range(_TNCH):
        xb, sem = xbufs[ci % 2]
        if ci - 2 >= 0:
            scats[ci - 2][0].wait()
            scats[ci - 2][1].wait()
        iv0 = i0_v[pl.ds(ci * _TCH, _TCH)]
        iv1 = i1_v[pl.ds(ci * _TCH, _TCH)]
        pltpu.sync_copy(src_hbm.at[pl.ds(base + ci * _TCH, _TCH)], xb)
        ca = pltpu.async_copy(xb, out_hbm.at[iv0], sem)
        cb = pltpu.async_copy(xb, out_hbm.at[iv1], sem)
        scats[ci] = (ca, cb)
    scats[_TNCH - 2][0].wait()
    scats[_TNCH - 2][1].wait()
    scats[_TNCH - 1][0].wait()
    scats[_TNCH - 1][1].wait()


@functools.cache
def _sc_dispatch():
    return pl.kernel(
        _sc_dispatch_body,
        out_type=jax.ShapeDtypeStruct((P, H), jnp.float32),
        mesh=plsc.VectorSubcoreMesh(core_axis_name="c", subcore_axis_name="s",
                                    num_cores=NC, num_subcores=NS),
        scratch_types=[
            pltpu.VMEM((S // NW,), jnp.int32),
            pltpu.VMEM((S // NW,), jnp.int32),
            pltpu.VMEM((_TCH, H), jnp.float32),
            pltpu.VMEM((_TCH, H), jnp.float32),
            pltpu.SemaphoreType.DMA,
            pltpu.SemaphoreType.DMA,
        ],
    )


def _sc_combine_body(y_hbm, i0_hbm, i1_hbm, w0_hbm, w1_hbm, out_hbm,
                     i0_v, i1_v, w0_v, w1_v, a0, b0, a1, b1,
                     gs0, gs1, ws0, ws1):
    wid = lax.axis_index("s") * NC + lax.axis_index("c")
    tok_per_w = S // NW
    base = wid * tok_per_w
    bufs = [(a0, b0, gs0, ws0), (a1, b1, gs1, ws1)]
    pltpu.sync_copy(i0_hbm.at[pl.ds(base, tok_per_w)], i0_v)
    pltpu.sync_copy(i1_hbm.at[pl.ds(base, tok_per_w)], i1_v)
    pltpu.sync_copy(w0_hbm.at[pl.ds(base * 16, tok_per_w * 16)], w0_v)
    pltpu.sync_copy(w1_hbm.at[pl.ds(base * 16, tok_per_w * 16)], w1_v)

    def issue(ci):
        av, bv, gs, _ = bufs[ci % 2]
        ca = pltpu.async_copy(y_hbm.at[i0_v[pl.ds(ci * _TCH, _TCH)]], av, gs)
        cb = pltpu.async_copy(y_hbm.at[i1_v[pl.ds(ci * _TCH, _TCH)]], bv, gs)
        return (ca, cb)

    gathers = [None] * _TNCH
    writes = [None] * _TNCH
    gathers[0] = issue(0)
    for ci in range(_TNCH):
        av, bv, _, ws = bufs[ci % 2]
        gathers[ci][0].wait()
        gathers[ci][1].wait()
        if ci + 1 < _TNCH:
            if ci - 1 >= 0:
                writes[ci - 1].wait()
            gathers[ci + 1] = issue(ci + 1)

        def addrow(r, carry, av=av, bv=bv, ci=ci):
            rbase = (ci * _TCH + r) * 16
            w0 = w0_v[pl.ds(rbase, 16)]
            w1 = w1_v[pl.ds(rbase, 16)]
            for c in range(H // 16):
                sl = pl.ds(c * 16, 16)
                av[r, sl] = w0 * av[r, sl] + w1 * bv[r, sl]
            return carry

        lax.fori_loop(0, _TCH, addrow, 0)
        writes[ci] = pltpu.async_copy(
            av, out_hbm.at[pl.ds(base + ci * _TCH, _TCH)], ws)
    writes[_TNCH - 2].wait()
    writes[_TNCH - 1].wait()


@functools.cache
def _sc_combine():
    return pl.kernel(
        _sc_combine_body,
        out_type=jax.ShapeDtypeStruct((S, H), jnp.float32),
        mesh=plsc.VectorSubcoreMesh(core_axis_name="c", subcore_axis_name="s",
                                    num_cores=NC, num_subcores=NS),
        scratch_types=[
            pltpu.VMEM((S // NW,), jnp.int32),
            pltpu.VMEM((S // NW,), jnp.int32),
            pltpu.VMEM((S // NW * 16,), jnp.float32),
            pltpu.VMEM((S // NW * 16,), jnp.float32),
            pltpu.VMEM((_TCH, H), jnp.float32),
            pltpu.VMEM((_TCH, H), jnp.float32),
            pltpu.VMEM((_TCH, H), jnp.float32),
            pltpu.VMEM((_TCH, H), jnp.float32),
            pltpu.SemaphoreType.DMA,
            pltpu.SemaphoreType.DMA,
            pltpu.SemaphoreType.DMA,
            pltpu.SemaphoreType.DMA,
        ],
    )


def _blk_expert(bc, i):
    e = jnp.int32(0)
    for k in range(E):
        e = e + (bc[k] <= i).astype(jnp.int32)
    return jnp.minimum(e, E - 1)


def _ffn1_body(bc_ref, x_ref, w1_ref, b1_ref, o_ref):
    i = pl.program_id(0)

    @pl.when(i < bc_ref[E - 1])
    def _():
        h = jnp.dot(x_ref[...].astype(jnp.bfloat16),
                    w1_ref[0].astype(jnp.bfloat16),
                    preferred_element_type=jnp.float32) + b1_ref[0]
        g = 0.5 * h * (1.0 + lax.erf(h * 0.7071067811865476))
        o_ref[...] = g.astype(jnp.bfloat16)


_ffn1 = pl.pallas_call(
    _ffn1_body,
    grid_spec=pltpu.PrefetchScalarGridSpec(
        num_scalar_prefetch=1,
        grid=(T,),
        in_specs=[
            pl.BlockSpec((BM, H), lambda i, bc: (i, 0)),
            pl.BlockSpec((1, H, F), lambda i, bc: (_blk_expert(bc, i), 0, 0)),
            pl.BlockSpec((1, 1, F), lambda i, bc: (_blk_expert(bc, i), 0, 0)),
        ],
        out_specs=pl.BlockSpec((BM, F), lambda i, bc: (i, 0)),
    ),
    out_shape=jax.ShapeDtypeStruct((P, F), jnp.bfloat16),
)


def _ffn2_body(bc_ref, h_ref, w2_ref, b2_ref, o_ref):
    i = pl.program_id(0)

    @pl.when(i < bc_ref[E - 1])
    def _():
        o_ref[...] = jnp.dot(h_ref[...], w2_ref[0].astype(jnp.bfloat16),
                             preferred_element_type=jnp.float32) + b2_ref[0]


_ffn2 = pl.pallas_call(
    _ffn2_body,
    grid_spec=pltpu.PrefetchScalarGridSpec(
        num_scalar_prefetch=1,
        grid=(T,),
        in_specs=[
            pl.BlockSpec((BM, F), lambda i, bc: (i, 0)),
            pl.BlockSpec((1, F, H), lambda i, bc: (_blk_expert(bc, i), 0, 0)),
            pl.BlockSpec((1, 1, H), lambda i, bc: (_blk_expert(bc, i), 0, 0)),
        ],
        out_specs=pl.BlockSpec((BM, H), lambda i, bc: (i, 0)),
    ),
    out_shape=jax.ShapeDtypeStruct((P, H), jnp.float32),
)


def kernel(x, Wg, W1, b1, W2, b2):
    x2d = x.reshape(S, H)
    wg_pad = jnp.zeros((H, 128), jnp.float32).at[:, :E].set(Wg)
    w_all, i_all = _router(x2d, wg_pad)
    pos_all, aux = _meta(i_all)
    bcum = aux[0, :E]

    i0 = pos_all[:, 0]
    i1 = pos_all[:, 1]
    w0x = jnp.broadcast_to(w_all[:, 0:1], (S, 16)).reshape(-1)
    w1x = jnp.broadcast_to(w_all[:, 1:2], (S, 16)).reshape(-1)

    x_sorted = _sc_dispatch()(x2d, i0, i1)
    h_act = _ffn1(bcum, x_sorted, W1, b1.reshape(E, 1, F))
    y = _ffn2(bcum, h_act, W2, b2.reshape(E, 1, H))
    out = _sc_combine()(y, i0, i1, w0x, w1x)
    return out.reshape(1, S, H)
